# baseline (speedup 1.0000x reference)
import jax
import jax.numpy as jnp
from jax import lax
from jax.experimental import pallas as pl
from jax.experimental.pallas import tpu as pltpu

HS = 2048
E = 16
DFF = 1024
NG = 4
GS = E // NG
TOPK = 2
RSF = 2.5
T = 2048
TILE = 128
NT = T // TILE


def _router_body(x_ref, w_ref, b_ref, cw_ref):
    x = x_ref[...]
    w = w_ref[...]
    logits = lax.dot_general(x, w, (((1,), (1,)), ((), ())),
                             preferred_element_type=jnp.float32)
    scores = jax.nn.sigmoid(logits)
    sfc = scores + b_ref[0:1, :]
    lane = lax.broadcasted_iota(jnp.int32, (TILE, E), 1)
    grp = lane // GS
    NEG = jnp.float32(-1e30)
    BIGI = jnp.int32(1 << 30)

    def top1(vals, mask):
        v = jnp.where(mask, vals, NEG)
        m = jnp.max(v, axis=1, keepdims=True)
        l = jnp.min(jnp.where((v == m) & mask, lane, BIGI), axis=1,
                    keepdims=True)
        return m, l

    gsum_lane = jnp.zeros((TILE, E), jnp.float32)
    for g in range(NG):
        gm = grp == g
        m1, l1 = top1(sfc, gm)
        m2, _ = top1(sfc, gm & (lane != l1))
        gsum_lane = jnp.where(gm, m1 + m2, gsum_lane)

    repr_mask = (lane % GS) == 0
    _, gl1 = top1(gsum_lane, repr_mask)
    _, gl2 = top1(gsum_lane, repr_mask & (lane != gl1))
    sel = (grp == gl1 // GS) | (grp == gl2 // GS)

    sfc2 = jnp.where(sel, sfc, 0.0)
    _, e0 = top1(sfc2, jnp.full((TILE, E), True))
    sfc3 = jnp.where(lane == e0, NEG, sfc2)
    _, e1 = top1(sfc3, jnp.full((TILE, E), True))

    w0 = jnp.sum(jnp.where(lane == e0, scores, 0.0), axis=1, keepdims=True)
    w1 = jnp.sum(jnp.where(lane == e1, scores, 0.0), axis=1, keepdims=True)
    denom = w0 + w1 + jnp.float32(1e-20)
    cw = (jnp.where(lane == e0, w0, 0.0) + jnp.where(lane == e1, w1, 0.0))
    cw_ref[...] = cw * (RSF / denom)


def _moe_body(cw_ref, x_ref, gw_ref, uw_ref, dw_ref, out_ref, acc_ref):
    e = pl.program_id(0)
    i = pl.program_id(1)
    x = x_ref[...]
    g = gw_ref[0]
    u = uw_ref[0]
    d = dw_ref[0]
    h1 = lax.dot_general(x, g, (((1,), (1,)), ((), ())),
                         preferred_element_type=jnp.float32)
    h2 = lax.dot_general(x, u, (((1,), (1,)), ((), ())),
                         preferred_element_type=jnp.float32)
    h = (h1 * jax.nn.sigmoid(h1) * h2).astype(jnp.bfloat16)
    y = lax.dot_general(h, d, (((1,), (1,)), ((), ())),
                        preferred_element_type=jnp.float32)
    lane = lax.broadcasted_iota(jnp.int32, (TILE, E), 1)
    coef = jnp.sum(jnp.where(lane == e, cw_ref[...], 0.0), axis=1,
                   keepdims=True)
    coef = jnp.where(e == E, 1.0, coef)
    contrib = y * coef

    @pl.when(e == 0)
    def _():
        acc_ref[pl.ds(i * TILE, TILE), :] = contrib

    @pl.when(e > 0)
    def _():
        acc_ref[pl.ds(i * TILE, TILE), :] += contrib

    @pl.when(e == E)
    def _():
        out_ref[...] = acc_ref[pl.ds(i * TILE, TILE), :]


def kernel(hidden_states, router_weight, gate_w, up_w, down_w,
           shared_gate_w, shared_up_w, shared_down_w, e_bias):
    x = hidden_states.reshape(T, HS)
    bias2d = jnp.broadcast_to(e_bias[None, :], (8, E))

    cw = pl.pallas_call(
        _router_body,
        grid=(NT,),
        in_specs=[
            pl.BlockSpec((TILE, HS), lambda i: (i, 0)),
            pl.BlockSpec((E, HS), lambda i: (0, 0)),
            pl.BlockSpec((8, E), lambda i: (0, 0)),
        ],
        out_specs=pl.BlockSpec((TILE, E), lambda i: (i, 0)),
        out_shape=jax.ShapeDtypeStruct((T, E), jnp.float32),
    )(x, router_weight, bias2d)

    gw_all = jnp.concatenate([gate_w, shared_gate_w[None]], axis=0)
    uw_all = jnp.concatenate([up_w, shared_up_w[None]], axis=0)
    dw_all = jnp.concatenate([down_w, shared_down_w[None]], axis=0)
    gw_all = gw_all.astype(jnp.bfloat16)
    uw_all = uw_all.astype(jnp.bfloat16)
    dw_all = dw_all.astype(jnp.bfloat16)
    xb = x.astype(jnp.bfloat16)

    out = pl.pallas_call(
        _moe_body,
        grid=(E + 1, NT),
        in_specs=[
            pl.BlockSpec((TILE, E), lambda e, i: (i, 0)),
            pl.BlockSpec((TILE, HS), lambda e, i: (i, 0)),
            pl.BlockSpec((1, DFF, HS), lambda e, i: (e, 0, 0)),
            pl.BlockSpec((1, DFF, HS), lambda e, i: (e, 0, 0)),
            pl.BlockSpec((1, HS, DFF), lambda e, i: (e, 0, 0)),
        ],
        out_specs=pl.BlockSpec((TILE, HS), lambda e, i: (i, 0)),
        out_shape=jax.ShapeDtypeStruct((T, HS), jnp.float32),
        scratch_shapes=[pltpu.VMEM((T, HS), jnp.float32)],
        compiler_params=pltpu.CompilerParams(
            dimension_semantics=("arbitrary", "arbitrary")),
    )(cw, xb, gw_all, uw_all, dw_all)

    return out.reshape(1, T, HS)


# speedup vs baseline: 1.5253x; 1.5253x over previous
import functools

import jax
import jax.numpy as jnp
from jax import lax
from jax.experimental import pallas as pl
from jax.experimental.pallas import tpu as pltpu
from jax.experimental.pallas import tpu_sc as plsc

HS = 2048
E = 16
DFF = 1024
NG = 4
GS = E // NG
RSF = 2.5
T = 2048
TILE = 128
NT = T // TILE

NC = 2
NS = 16
NW = NC * NS
TPW = T // NW
NRT = 48
PR = NRT * TILE
NTG = NT + NRT
YROWS = NTG * TILE
HSI = HS // 2


def _router_body(x_ref, w_ref, b_ref, cw_ref):
    x = x_ref[...]
    w = w_ref[...]
    logits = lax.dot_general(x, w, (((1,), (1,)), ((), ())),
                             preferred_element_type=jnp.float32)
    scores = jax.nn.sigmoid(logits)
    sfc = scores + b_ref[0:1, :]
    lane = lax.broadcasted_iota(jnp.int32, (TILE, E), 1)
    grp = lane // GS
    NEG = jnp.float32(-1e30)
    BIGI = jnp.int32(1 << 30)

    def top1(vals, mask):
        v = jnp.where(mask, vals, NEG)
        m = jnp.max(v, axis=1, keepdims=True)
        l = jnp.min(jnp.where((v == m) & mask, lane, BIGI), axis=1,
                    keepdims=True)
        return m, l

    gsum_lane = jnp.zeros((TILE, E), jnp.float32)
    for g in range(NG):
        gm = grp == g
        m1, l1 = top1(sfc, gm)
        m2, _ = top1(sfc, gm & (lane != l1))
        gsum_lane = jnp.where(gm, m1 + m2, gsum_lane)

    repr_mask = (lane % GS) == 0
    _, gl1 = top1(gsum_lane, repr_mask)
    _, gl2 = top1(gsum_lane, repr_mask & (lane != gl1))
    sel = (grp == gl1 // GS) | (grp == gl2 // GS)

    sfc2 = jnp.where(sel, sfc, 0.0)
    _, e0 = top1(sfc2, jnp.full((TILE, E), True))
    sfc3 = jnp.where(lane == e0, NEG, sfc2)
    _, e1 = top1(sfc3, jnp.full((TILE, E), True))

    w0 = jnp.sum(jnp.where(lane == e0, scores, 0.0), axis=1, keepdims=True)
    w1 = jnp.sum(jnp.where(lane == e1, scores, 0.0), axis=1, keepdims=True)
    denom = w0 + w1 + jnp.float32(1e-20)
    cw = (jnp.where(lane == e0, w0, 0.0) + jnp.where(lane == e1, w1, 0.0))
    cw_ref[...] = cw * (RSF / denom)


_sc_mesh = plsc.VectorSubcoreMesh(core_axis_name="c", subcore_axis_name="s",
                                  num_cores=NC, num_subcores=NS)


@functools.partial(
    pl.kernel,
    out_type=(
        jax.ShapeDtypeStruct((T,), jnp.int32),
        jax.ShapeDtypeStruct((T,), jnp.int32),
        jax.ShapeDtypeStruct((T,), jnp.float32),
        jax.ShapeDtypeStruct((T,), jnp.float32),
        jax.ShapeDtypeStruct((E,), jnp.int32),
        jax.ShapeDtypeStruct((PR + TILE, HSI), jnp.int32),
    ),
    mesh=_sc_mesh,
    scratch_types=[
        pltpu.VMEM((T * E,), jnp.float32),
        pltpu.VMEM((TPW, HSI), jnp.int32),
        pltpu.VMEM((TPW,), jnp.int32),
        pltpu.VMEM((TPW,), jnp.int32),
        pltpu.VMEM((TPW,), jnp.int32),
        pltpu.VMEM((TPW,), jnp.int32),
        pltpu.VMEM((TPW,), jnp.float32),
        pltpu.VMEM((TPW,), jnp.float32),
        pltpu.VMEM((E,), jnp.int32),
        pltpu.VMEM((E,), jnp.int32),
        pltpu.VMEM((E,), jnp.int32),
        pltpu.SemaphoreType.DMA,
    ],
    compiler_params=pltpu.CompilerParams(needs_layout_passes=False),
)
def _dispatch(cw_hbm, xi_hbm, pos0_hbm, pos1_hbm, w0_hbm, w1_hbm, cnt_hbm,
              xs_hbm, cwv, xrows, p0v, p1v, p0sv, p1sv, w0v, w1v, cntv, runv,
              tmpv, sem):
    c = lax.axis_index("c")
    s = lax.axis_index("s")
    wid = s * NC + c
    base = wid * TPW
    pltpu.sync_copy(cw_hbm, cwv)
    pltpu.sync_copy(xi_hbm.at[pl.ds(base, TPW)], xrows)
    iota = lax.iota(jnp.int32, 16)
    zeros_i = jnp.zeros((E,), jnp.int32)
    ones_i = jnp.ones((E,), jnp.int32)
    zeros_f = jnp.zeros((E,), jnp.float32)
    bchunk = wid * (TPW // 16)

    def cnt_body(ch, carry):
        tot, pref = carry
        ssum = zeros_i
        for k in range(16):
            row = cwv[pl.ds(ch * (16 * E) + k * E, E)]
            ssum = ssum + jnp.where(row > zeros_f, ones_i, zeros_i)
        tot = tot + ssum
        inpref = jnp.full((E,), ch, jnp.int32) < jnp.full((E,), bchunk,
                                                         jnp.int32)
        pref = pref + jnp.where(inpref, ssum, zeros_i)
        return tot, pref

    tot, pref = lax.fori_loop(0, T // 16, cnt_body, (zeros_i, zeros_i))
    nb = lax.shift_right_logical(tot + jnp.full((E,), TILE - 1, jnp.int32),
                                 jnp.full((E,), 7, jnp.int32))
    cumnb = nb
    for d in (1, 2, 4, 8):
        tmpv[...] = cumnb
        dv = jnp.full((E,), d, jnp.int32)
        sh = plsc.load_gather(tmpv, [(iota - dv) & jnp.full((E,), 15,
                                                           jnp.int32)])
        cumnb = cumnb + jnp.where(iota >= dv, sh, zeros_i)
    start = (cumnb - nb) * jnp.full((E,), TILE, jnp.int32) + pref

    @pl.when(wid == 0)
    def _():
        cntv[...] = tot
        pltpu.sync_copy(cntv, cnt_hbm)

    trash = jnp.full((E,), PR, jnp.int32)
    fifteen = jnp.full((E,), 15, jnp.int32)
    twos_i = jnp.full((E,), 2, jnp.int32)
    runv[...] = start
    for g in range(TPW // 16):
        gb = base + g * 16

        def pos_body(t, carry):
            p0a, p1a, p0sa, p1sa, w0a, w1a = carry
            rowbase = (gb + t) * E
            rowbase_v = jnp.full((E,), rowbase, jnp.int32)
            row = cwv[pl.ds(rowbase, E)]
            m = row > zeros_f
            mi = jnp.where(m, ones_i, zeros_i)
            nselv = plsc.all_reduce_population_count(m)
            e0s = plsc.all_reduce_ffs(m) & fifteen
            m1 = m & (iota != e0s)
            e1s = plsc.all_reduce_ffs(m1) & fifteen
            p0 = plsc.load_gather(runv, [e0s])
            p1 = plsc.load_gather(runv, [e1s])
            w0s = plsc.load_gather(cwv, [rowbase_v + e0s])
            w1s = plsc.load_gather(cwv, [rowbase_v + e1s])
            w0s = jnp.where(nselv >= ones_i, w0s, zeros_f)
            w1s = jnp.where(nselv >= twos_i, w1s, zeros_f)
            slot = iota == jnp.full((E,), t, jnp.int32)
            p0a = jnp.where(slot, p0, p0a)
            p1a = jnp.where(slot, p1, p1a)
            w0a = jnp.where(slot, w0s, w0a)
            w1a = jnp.where(slot, w1s, w1a)
            p0sa = jnp.where(slot & (nselv >= ones_i), p0, p0sa)
            p1sa = jnp.where(slot & (nselv >= twos_i), p1, p1sa)
            runv[...] = runv[...] + mi
            return p0a, p1a, p0sa, p1sa, w0a, w1a

        p0a, p1a, p0sa, p1sa, w0a, w1a = lax.fori_loop(
            0, 16, pos_body,
            (zeros_i, zeros_i, trash, trash, zeros_f, zeros_f))
        sl = pl.ds(g * 16, 16)
        p0v[sl] = p0a
        p1v[sl] = p1a
        p0sv[sl] = p0sa
        p1sv[sl] = p1sa
        w0v[sl] = w0a
        w1v[sl] = w1a

    pltpu.sync_copy(p0v, pos0_hbm.at[pl.ds(base, TPW)])
    pltpu.sync_copy(p1v, pos1_hbm.at[pl.ds(base, TPW)])
    pltpu.sync_copy(w0v, w0_hbm.at[pl.ds(base, TPW)])
    pltpu.sync_copy(w1v, w1_hbm.at[pl.ds(base, TPW)])
    pltpu.async_copy(xrows, xs_hbm.at[p0sv], sem).wait()
    pltpu.async_copy(xrows, xs_hbm.at[p1sv], sem).wait()


def _gemm_body(widx_ref, valid_ref, xb_ref, xs_ref, gw_ref, uw_ref, dw_ref,
               ys_ref):
    j = pl.program_id(0)

    @pl.when(valid_ref[j] == 1)
    def _():
        xsel = jnp.where(j < NT, xb_ref[...], xs_ref[...])
        g = gw_ref[0]
        u = uw_ref[0]
        d = dw_ref[0]
        h1 = lax.dot_general(xsel, g, (((1,), (1,)), ((), ())),
                             preferred_element_type=jnp.float32)
        h2 = lax.dot_general(xsel, u, (((1,), (1,)), ((), ())),
                             preferred_element_type=jnp.float32)
        h = (h1 * jax.nn.sigmoid(h1) * h2).astype(jnp.bfloat16)
        ys_ref[...] = lax.dot_general(h, d, (((1,), (1,)), ((), ())),
                                      preferred_element_type=jnp.float32)


@functools.partial(
    pl.kernel,
    out_type=jax.ShapeDtypeStruct((T, HS), jnp.float32),
    mesh=_sc_mesh,
    scratch_types=[
        pltpu.VMEM((16, HS), jnp.float32),
        pltpu.VMEM((16, HS), jnp.float32),
        pltpu.VMEM((16, HS), jnp.float32),
        pltpu.VMEM((TPW // 16, 16), jnp.int32),
        pltpu.VMEM((TPW // 16, 16), jnp.int32),
        pltpu.VMEM((TPW // 16, 16), jnp.float32),
        pltpu.VMEM((TPW // 16, 16), jnp.float32),
        pltpu.SemaphoreType.DMA,
        pltpu.SemaphoreType.DMA,
        pltpu.SemaphoreType.DMA,
    ],
    compiler_params=pltpu.CompilerParams(needs_layout_passes=False),
)
def _combine(ys_hbm, idx0_hbm, idx1_hbm, w0_hbm, w1_hbm, out_hbm,
             shv, r0v, r1v, i0v, i1v, wv0, wv1, sem, sem0, sem1):
    c = lax.axis_index("c")
    s = lax.axis_index("s")
    wid = s * NC + c
    base = wid * TPW
    iota = lax.iota(jnp.int32, 16)
    pltpu.sync_copy(idx0_hbm.at[wid], i0v)
    pltpu.sync_copy(idx1_hbm.at[wid], i1v)
    pltpu.sync_copy(w0_hbm.at[wid], wv0)
    pltpu.sync_copy(w1_hbm.at[wid], wv1)
    def gbody(g, _):
        gb = base + g * 16
        a_sh = pltpu.async_copy(ys_hbm.at[pl.ds(gb, 16)], shv, sem)
        i0row = i0v[g]
        i1row = i1v[g]
        descs = []
        for k in range(16):
            descs.append(pltpu.async_copy(
                ys_hbm.at[pl.ds(i0row[k], 1)], r0v.at[pl.ds(k, 1)], sem0))
            descs.append(pltpu.async_copy(
                ys_hbm.at[pl.ds(i1row[k], 1)], r1v.at[pl.ds(k, 1)], sem1))
        a_sh.wait()
        for a in descs:
            a.wait()
        w0row = wv0[g]
        w1row = wv1[g]
        for k in range(16):
            w0s = jnp.full((16,), w0row[k], jnp.float32)
            w1s = jnp.full((16,), w1row[k], jnp.float32)

            def vbody(v, _, k=k, w0s=w0s, w1s=w1s):
                b = v * 64
                for q in range(4):
                    sl = pl.ds(b + q * 16, 16)
                    shv[k, sl] = (shv[k, sl] + w0s * r0v[k, sl]
                                  + w1s * r1v[k, sl])
                return 0

            lax.fori_loop(0, HS // 64, vbody, 0)
        pltpu.sync_copy(shv, out_hbm.at[pl.ds(gb, 16)])
        return 0

    lax.fori_loop(0, TPW // 16, gbody, 0)


def kernel(hidden_states, router_weight, gate_w, up_w, down_w,
           shared_gate_w, shared_up_w, shared_down_w, e_bias):
    x = hidden_states.reshape(T, HS)
    bias2d = jnp.broadcast_to(e_bias[None, :], (8, E))

    cw = pl.pallas_call(
        _router_body,
        grid=(NT,),
        in_specs=[
            pl.BlockSpec((TILE, HS), lambda i: (i, 0)),
            pl.BlockSpec((E, HS), lambda i: (0, 0)),
            pl.BlockSpec((8, E), lambda i: (0, 0)),
        ],
        out_specs=pl.BlockSpec((TILE, E), lambda i: (i, 0)),
        out_shape=jax.ShapeDtypeStruct((T, E), jnp.float32),
    )(x, router_weight, bias2d)

    xb = x.astype(jnp.bfloat16)
    xi = lax.bitcast_convert_type(xb.reshape(T, HSI, 2), jnp.int32)

    pos0, pos1, w0, w1, counts, xsi = _dispatch(cw.reshape(T * E), xi)

    nb = (counts + TILE - 1) // TILE
    cum = jnp.cumsum(nb)
    nvalid = cum[E - 1]
    j48 = jnp.arange(NRT, dtype=jnp.int32)
    rexp = jnp.sum((cum[None, :] <= j48[:, None]).astype(jnp.int32), axis=1)
    widx = jnp.concatenate([
        jnp.full((NT,), E, jnp.int32),
        jnp.where(j48 < nvalid, rexp, E - 1).astype(jnp.int32),
    ])
    valid = jnp.concatenate([
        jnp.ones((NT,), jnp.int32),
        (j48 < nvalid).astype(jnp.int32),
    ])

    xs = lax.bitcast_convert_type(xsi, jnp.bfloat16).reshape(PR + TILE, HS)
    gw_all = jnp.concatenate([gate_w, shared_gate_w[None]], 0).astype(
        jnp.bfloat16)
    uw_all = jnp.concatenate([up_w, shared_up_w[None]], 0).astype(
        jnp.bfloat16)
    dw_all = jnp.concatenate([down_w, shared_down_w[None]], 0).astype(
        jnp.bfloat16)

    ys = pl.pallas_call(
        _gemm_body,
        grid_spec=pltpu.PrefetchScalarGridSpec(
            num_scalar_prefetch=2,
            grid=(NTG,),
            in_specs=[
                pl.BlockSpec((TILE, HS),
                             lambda j, wr, vr: (jnp.where(j < NT, j, 0), 0)),
                pl.BlockSpec((TILE, HS),
                             lambda j, wr, vr: (jnp.where(j >= NT, j - NT, 0),
                                                0)),
                pl.BlockSpec((1, DFF, HS), lambda j, wr, vr: (wr[j], 0, 0)),
                pl.BlockSpec((1, DFF, HS), lambda j, wr, vr: (wr[j], 0, 0)),
                pl.BlockSpec((1, HS, DFF), lambda j, wr, vr: (wr[j], 0, 0)),
            ],
            out_specs=pl.BlockSpec((TILE, HS), lambda j, wr, vr: (j, 0)),
        ),
        out_shape=jax.ShapeDtypeStruct((YROWS, HS), jnp.float32),
        compiler_params=pltpu.CompilerParams(
            dimension_semantics=("arbitrary",)),
    )(widx, valid, xb, xs, gw_all, uw_all, dw_all)

    ar = jnp.arange(T, dtype=jnp.int32)
    idx0 = jnp.where(w0 > 0, pos0 + T, ar).reshape(NW, TPW // 16, 16)
    idx1 = jnp.where(w1 > 0, pos1 + T, ar).reshape(NW, TPW // 16, 16)
    out = _combine(ys, idx0, idx1, w0.reshape(NW, TPW // 16, 16),
                   w1.reshape(NW, TPW // 16, 16))
    return out.reshape(1, T, HS)
